# baseline (speedup 1.0000x reference)
import jax
import jax.numpy as jnp
from jax import lax
from jax.experimental import pallas as pl
from jax.experimental.pallas import tpu as pltpu
from jax.experimental.pallas import tpu_sc as plsc

_NUM_VOICES = 1000
_MAX_LENGTH = 510
_STYLE_DIM = 256
_BATCH = 16384

_NC = 2
_NS = 16
_LANES = 16
_NW = _NC * _NS
_BPW = _BATCH // _NW

_SIZES = (16, 32, 64, 96, 96, 96, 64, 32, 16)
_OFFS = tuple(sum(_SIZES[:i]) for i in range(len(_SIZES)))
_NCHUNK = len(_SIZES)
_NB = 5
_BUF_ROWS = max(_SIZES)


def _body(vid_hbm, len_hbm, tab_hbm, out_hbm,
          vid_v, len_v, idx_v, rows_v,
          si0, si1, gs0, gs1, gs2, gs3, gs4, os0, os1, os2, os3, os4):
    wid = lax.axis_index("s") * _NC + lax.axis_index("c")
    base = wid * _BPW

    cin_v = pltpu.async_copy(vid_hbm.at[pl.ds(base, _BPW)], vid_v, si0)
    cin_l = pltpu.async_copy(len_hbm.at[pl.ds(base, _BPW)], len_v, si1)
    cin_v.wait()
    cin_l.wait()

    gsems = (gs0, gs1, gs2, gs3, gs4)
    osems = (os0, os1, os2, os3, os4)

    def compute_idx(c):
        for j in range(_SIZES[c] // _LANES):
            o = _OFFS[c] + j * _LANES
            v = vid_v[pl.ds(o, _LANES)]
            l = len_v[pl.ds(o, _LANES)]
            idx_v[pl.ds(o, _LANES)] = (
                (jnp.clip(l, 1, _MAX_LENGTH) - 1) * _NUM_VOICES + v)

    def fire_gather(c):
        o, n = _OFFS[c], _SIZES[c]
        return pltpu.async_copy(
            tab_hbm.at[idx_v.at[pl.ds(o, n)]],
            rows_v.at[c % _NB, pl.ds(0, n)], gsems[c % _NB])

    def fire_out(c):
        o, n = _OFFS[c], _SIZES[c]
        return pltpu.async_copy(
            rows_v.at[c % _NB, pl.ds(0, n)],
            out_hbm.at[pl.ds(base + o, n)], osems[c % _NB])

    g = [None] * _NCHUNK
    o = [None] * _NCHUNK
    for c in range(_NB):
        compute_idx(c)
        g[c] = fire_gather(c)
    for c in range(_NB, _NCHUNK):
        compute_idx(c)
    for c in range(_NCHUNK):
        g[c].wait()
        o[c] = fire_out(c)
        if c + _NB < _NCHUNK:
            o[c].wait()
            g[c + _NB] = fire_gather(c + _NB)
    for c in range(max(0, _NCHUNK - _NB), _NCHUNK):
        o[c].wait()


@jax.jit
def _lookup(voice_ids, phoneme_lengths, table_flat):
    mesh = plsc.VectorSubcoreMesh(core_axis_name="c", subcore_axis_name="s")
    f = pl.kernel(
        _body,
        out_type=jax.ShapeDtypeStruct((_BATCH, _STYLE_DIM), jnp.float32),
        mesh=mesh,
        scratch_types=[
            pltpu.VMEM((_BPW,), jnp.int32),
            pltpu.VMEM((_BPW,), jnp.int32),
            pltpu.VMEM((_BPW,), jnp.int32),
            pltpu.VMEM((_NB, _BUF_ROWS, _STYLE_DIM), jnp.float32),
        ] + [pltpu.SemaphoreType.DMA] * 12,
    )
    return f(voice_ids, phoneme_lengths, table_flat)


def kernel(voice_ids, phoneme_lengths, table):
    table_flat = table.transpose(1, 0, 2).reshape(
        _MAX_LENGTH * _NUM_VOICES, _STYLE_DIM)
    return _lookup(voice_ids.astype(jnp.int32),
                   phoneme_lengths.astype(jnp.int32), table_flat)


# speedup vs baseline: 1.0219x; 1.0219x over previous
import jax
import jax.numpy as jnp
from jax import lax
from jax.experimental import pallas as pl
from jax.experimental.pallas import tpu as pltpu
from jax.experimental.pallas import tpu_sc as plsc

_NUM_VOICES = 1000
_MAX_LENGTH = 510
_STYLE_DIM = 256
_BATCH = 16384

_NC = 2
_NS = 16
_LANES = 16
_NW = _NC * _NS
_BPW = _BATCH // _NW

_SIZES = (32, 96, 96, 96, 96, 64, 32)
_OFFS = tuple(sum(_SIZES[:i]) for i in range(len(_SIZES)))
_NCHUNK = len(_SIZES)
_NB = 5
_BUF_ROWS = max(_SIZES)


def _body(vid_hbm, len_hbm, tab_hbm, out_hbm,
          vid_v, len_v, idx_v, rows_v,
          si0, si1, gs0, gs1, gs2, gs3, gs4, os0, os1, os2, os3, os4):
    wid = lax.axis_index("s") * _NC + lax.axis_index("c")
    base = wid * _BPW

    cin_v = pltpu.async_copy(vid_hbm.at[pl.ds(base, _BPW)], vid_v, si0)
    cin_l = pltpu.async_copy(len_hbm.at[pl.ds(base, _BPW)], len_v, si1)
    cin_v.wait()
    cin_l.wait()

    gsems = (gs0, gs1, gs2, gs3, gs4)
    osems = (os0, os1, os2, os3, os4)

    def compute_idx(c):
        for j in range(_SIZES[c] // _LANES):
            o = _OFFS[c] + j * _LANES
            v = vid_v[pl.ds(o, _LANES)]
            l = len_v[pl.ds(o, _LANES)]
            idx_v[pl.ds(o, _LANES)] = (
                (jnp.clip(l, 1, _MAX_LENGTH) - 1) * _NUM_VOICES + v)

    def fire_gather(c):
        o, n = _OFFS[c], _SIZES[c]
        return pltpu.async_copy(
            tab_hbm.at[idx_v.at[pl.ds(o, n)]],
            rows_v.at[c % _NB, pl.ds(0, n)], gsems[c % _NB])

    def fire_out(c):
        o, n = _OFFS[c], _SIZES[c]
        return pltpu.async_copy(
            rows_v.at[c % _NB, pl.ds(0, n)],
            out_hbm.at[pl.ds(base + o, n)], osems[c % _NB])

    g = [None] * _NCHUNK
    o = [None] * _NCHUNK
    for c in range(_NB):
        compute_idx(c)
        g[c] = fire_gather(c)
    for c in range(_NB, _NCHUNK):
        compute_idx(c)
    for c in range(_NCHUNK):
        g[c].wait()
        o[c] = fire_out(c)
        if c + _NB < _NCHUNK:
            o[c].wait()
            g[c + _NB] = fire_gather(c + _NB)
    for c in range(max(0, _NCHUNK - _NB), _NCHUNK):
        o[c].wait()


@jax.jit
def _lookup(voice_ids, phoneme_lengths, table_flat):
    mesh = plsc.VectorSubcoreMesh(core_axis_name="c", subcore_axis_name="s")
    f = pl.kernel(
        _body,
        out_type=jax.ShapeDtypeStruct((_BATCH, _STYLE_DIM), jnp.float32),
        mesh=mesh,
        scratch_types=[
            pltpu.VMEM((_BPW,), jnp.int32),
            pltpu.VMEM((_BPW,), jnp.int32),
            pltpu.VMEM((_BPW,), jnp.int32),
            pltpu.VMEM((_NB, _BUF_ROWS, _STYLE_DIM), jnp.float32),
        ] + [pltpu.SemaphoreType.DMA] * 12,
    )
    return f(voice_ids, phoneme_lengths, table_flat)


def kernel(voice_ids, phoneme_lengths, table):
    table_flat = table.transpose(1, 0, 2).reshape(
        _MAX_LENGTH * _NUM_VOICES, _STYLE_DIM)
    return _lookup(voice_ids.astype(jnp.int32),
                   phoneme_lengths.astype(jnp.int32), table_flat)


# speedup vs baseline: 1.0280x; 1.0060x over previous
import jax
import jax.numpy as jnp
from jax import lax
from jax.experimental import pallas as pl
from jax.experimental.pallas import tpu as pltpu
from jax.experimental.pallas import tpu_sc as plsc

_NUM_VOICES = 1000
_MAX_LENGTH = 510
_STYLE_DIM = 256
_BATCH = 16384

_NC = 2
_NS = 16
_LANES = 16
_NW = _NC * _NS
_BPW = _BATCH // _NW

_SIZES = (32, 96, 96, 96, 96, 64, 32)
_OFFS = tuple(sum(_SIZES[:i]) for i in range(len(_SIZES)))
_NCHUNK = len(_SIZES)
_NB = 4
_BUF_ROWS = max(_SIZES)


def _body(vid_hbm, len_hbm, tab_hbm, out_hbm,
          vid_v, len_v, idx_v, rows_v,
          si0, si1, gs0, gs1, gs2, gs3, os0, os1, os2, os3):
    wid = lax.axis_index("s") * _NC + lax.axis_index("c")
    base = wid * _BPW

    cin_v = pltpu.async_copy(vid_hbm.at[pl.ds(base, _BPW)], vid_v, si0)
    cin_l = pltpu.async_copy(len_hbm.at[pl.ds(base, _BPW)], len_v, si1)
    cin_v.wait()
    cin_l.wait()

    gsems = (gs0, gs1, gs2, gs3)
    osems = (os0, os1, os2, os3)

    def compute_idx(c):
        for j in range(_SIZES[c] // _LANES):
            o = _OFFS[c] + j * _LANES
            v = vid_v[pl.ds(o, _LANES)]
            l = len_v[pl.ds(o, _LANES)]
            idx_v[pl.ds(o, _LANES)] = (
                (jnp.clip(l, 1, _MAX_LENGTH) - 1) * _NUM_VOICES + v)

    def fire_gather(c):
        o, n = _OFFS[c], _SIZES[c]
        return pltpu.async_copy(
            tab_hbm.at[idx_v.at[pl.ds(o, n)]],
            rows_v.at[c % _NB, pl.ds(0, n)], gsems[c % _NB])

    def fire_out(c):
        o, n = _OFFS[c], _SIZES[c]
        return pltpu.async_copy(
            rows_v.at[c % _NB, pl.ds(0, n)],
            out_hbm.at[pl.ds(base + o, n)], osems[c % _NB])

    g = [None] * _NCHUNK
    o = [None] * _NCHUNK
    for c in range(_NB):
        compute_idx(c)
        g[c] = fire_gather(c)
    for c in range(_NB, _NCHUNK):
        compute_idx(c)
    for c in range(_NCHUNK):
        g[c].wait()
        o[c] = fire_out(c)
        if c + _NB < _NCHUNK:
            o[c].wait()
            g[c + _NB] = fire_gather(c + _NB)
    for c in range(max(0, _NCHUNK - _NB), _NCHUNK):
        o[c].wait()


@jax.jit
def _lookup(voice_ids, phoneme_lengths, table_flat):
    mesh = plsc.VectorSubcoreMesh(core_axis_name="c", subcore_axis_name="s")
    f = pl.kernel(
        _body,
        out_type=jax.ShapeDtypeStruct((_BATCH, _STYLE_DIM), jnp.float32),
        mesh=mesh,
        scratch_types=[
            pltpu.VMEM((_BPW,), jnp.int32),
            pltpu.VMEM((_BPW,), jnp.int32),
            pltpu.VMEM((_BPW,), jnp.int32),
            pltpu.VMEM((_NB, _BUF_ROWS, _STYLE_DIM), jnp.float32),
        ] + [pltpu.SemaphoreType.DMA] * 10,
    )
    return f(voice_ids, phoneme_lengths, table_flat)


def kernel(voice_ids, phoneme_lengths, table):
    table_flat = table.transpose(1, 0, 2).reshape(
        _MAX_LENGTH * _NUM_VOICES, _STYLE_DIM)
    return _lookup(voice_ids.astype(jnp.int32),
                   phoneme_lengths.astype(jnp.int32), table_flat)


# speedup vs baseline: 1.0289x; 1.0008x over previous
import jax
import jax.numpy as jnp
from jax import lax
from jax.experimental import pallas as pl
from jax.experimental.pallas import tpu as pltpu
from jax.experimental.pallas import tpu_sc as plsc

_NUM_VOICES = 1000
_MAX_LENGTH = 510
_STYLE_DIM = 256
_BATCH = 16384

_NC = 2
_NS = 16
_LANES = 16
_NW = _NC * _NS
_BPW = _BATCH // _NW

_SIZES = (32, 96, 96, 96, 96, 64, 32)
_OFFS = tuple(sum(_SIZES[:i]) for i in range(len(_SIZES)))
_NCHUNK = len(_SIZES)
_NB = 4
_BUF_ROWS = max(_SIZES)


def _body(vid_hbm, len_hbm, tab_hbm, out_hbm,
          vid_v, len_v, idx_v, rows_v,
          si0, si1, gs0, gs1, gs2, gs3, os0, os1, os2, os3):
    wid = lax.axis_index("s") * _NC + lax.axis_index("c")
    base = wid * _BPW

    cin_v = pltpu.async_copy(vid_hbm.at[pl.ds(base, _BPW)], vid_v, si0)
    cin_l = pltpu.async_copy(len_hbm.at[pl.ds(base, _BPW)], len_v, si1)
    cin_v.wait()
    cin_l.wait()

    gsems = (gs0, gs1, gs2, gs3)
    osems = (os0, os1, os2, os3)

    def compute_idx(c):
        for j in range(_SIZES[c] // _LANES):
            o = _OFFS[c] + j * _LANES
            v = vid_v[pl.ds(o, _LANES)]
            l = len_v[pl.ds(o, _LANES)]
            idx_v[pl.ds(o, _LANES)] = (
                (jnp.clip(l, 1, _MAX_LENGTH) - 1) * _NUM_VOICES + v)

    def fire_gather(c):
        o, n = _OFFS[c], _SIZES[c]
        return pltpu.async_copy(
            tab_hbm.at[idx_v.at[pl.ds(o, n)]],
            rows_v.at[c % _NB, pl.ds(0, n)], gsems[c % _NB])

    def fire_out(c):
        o, n = _OFFS[c], _SIZES[c]
        return pltpu.async_copy(
            rows_v.at[c % _NB, pl.ds(0, n)],
            out_hbm.at[pl.ds(base + o, n)], osems[c % _NB])

    g = [None] * _NCHUNK
    o = [None] * _NCHUNK
    compute_idx(0)
    g[0] = fire_gather(0)
    compute_idx(1)
    g[1] = fire_gather(1)

    def idx_body(i, carry):
        off = pl.multiple_of(_OFFS[2] + i * _LANES, _LANES)
        v = vid_v[pl.ds(off, _LANES)]
        l = len_v[pl.ds(off, _LANES)]
        idx_v[pl.ds(off, _LANES)] = (
            (jnp.clip(l, 1, _MAX_LENGTH) - 1) * _NUM_VOICES + v)
        return carry

    lax.fori_loop(0, (_BPW - _OFFS[2]) // _LANES, idx_body, 0)
    for c in range(2, _NB):
        g[c] = fire_gather(c)
    for c in range(_NCHUNK):
        g[c].wait()
        o[c] = fire_out(c)
        if c + _NB < _NCHUNK:
            o[c].wait()
            g[c + _NB] = fire_gather(c + _NB)
    for c in range(max(0, _NCHUNK - _NB), _NCHUNK):
        o[c].wait()


@jax.jit
def _lookup(voice_ids, phoneme_lengths, table_flat):
    mesh = plsc.VectorSubcoreMesh(core_axis_name="c", subcore_axis_name="s")
    f = pl.kernel(
        _body,
        out_type=jax.ShapeDtypeStruct((_BATCH, _STYLE_DIM), jnp.float32),
        mesh=mesh,
        scratch_types=[
            pltpu.VMEM((_BPW,), jnp.int32),
            pltpu.VMEM((_BPW,), jnp.int32),
            pltpu.VMEM((_BPW,), jnp.int32),
            pltpu.VMEM((_NB, _BUF_ROWS, _STYLE_DIM), jnp.float32),
        ] + [pltpu.SemaphoreType.DMA] * 10,
    )
    return f(voice_ids, phoneme_lengths, table_flat)


def kernel(voice_ids, phoneme_lengths, table):
    table_flat = table.transpose(1, 0, 2).reshape(
        _MAX_LENGTH * _NUM_VOICES, _STYLE_DIM)
    return _lookup(voice_ids.astype(jnp.int32),
                   phoneme_lengths.astype(jnp.int32), table_flat)
